# Initial kernel scaffold; baseline (speedup 1.0000x reference)
import jax
import jax.numpy as jnp
from jax.experimental import pallas as pl


def kernel(indices, values):
    raise NotImplementedError("write your pallas kernel here")


# speedup vs baseline: 20.8278x; 20.8278x over previous
import functools

import jax
import jax.numpy as jnp
from jax import lax
from jax.experimental import pallas as pl
from jax.experimental.pallas import tpu as pltpu
from jax.experimental.pallas import tpu_sc as plsc

B = 4096
NNZ = 256
M = 2048
L = 16

NC = 2
NS = 16
NW = NC * NS
ROWS = B // NW


def _body(idx_hbm, val_hbm, out_hbm, idx_v, val_v, dense_v):
    c = lax.axis_index("c")
    s = lax.axis_index("s")
    wid = s * NC + c
    base = wid * ROWS
    pltpu.sync_copy(idx_hbm.at[pl.ds(base, ROWS)], idx_v)
    pltpu.sync_copy(val_hbm.at[pl.ds(base, ROWS)], val_v)

    zeros16 = jnp.zeros((L,), jnp.float32)

    def row(r, carry):
        def zero(i, c2):
            dense_v[pl.ds(i * L, L)] = zeros16
            return c2
        lax.fori_loop(0, M // L, zero, 0)
        for q in range(NNZ // L):
            idx16 = idx_v[r, pl.ds(q * L, L)]
            val16 = val_v[r, pl.ds(q * L, L)]
            plsc.addupdate_scatter(dense_v, [idx16], val16)
        pltpu.sync_copy(dense_v, out_hbm.at[base + r])
        return carry

    lax.fori_loop(0, ROWS, row, 0)


_sc_call = functools.partial(
    pl.kernel,
    mesh=plsc.VectorSubcoreMesh(core_axis_name="c", subcore_axis_name="s"),
    out_type=jax.ShapeDtypeStruct((B, M), jnp.float32),
    compiler_params=pltpu.CompilerParams(needs_layout_passes=False),
    scratch_types=[
        pltpu.VMEM((ROWS, NNZ), jnp.int32),
        pltpu.VMEM((ROWS, NNZ), jnp.float32),
        pltpu.VMEM((M,), jnp.float32),
    ],
)(_body)


def kernel(indices, values):
    return _sc_call(indices, values)


# speedup vs baseline: 42.8987x; 2.0597x over previous
import functools

import jax
import jax.numpy as jnp
from jax import lax
from jax.experimental import pallas as pl
from jax.experimental.pallas import tpu as pltpu
from jax.experimental.pallas import tpu_sc as plsc

B = 4096
NNZ = 256
M = 2048
L = 16

NC = 2
NS = 16
NW = NC * NS
ROWS = B // NW
G = 8
NG = ROWS // G
NQ = NNZ // L


def _body(idx_hbm, val_hbm, out_hbm, idx_v, val_v, dense_v, sem0, sem1):
    c = lax.axis_index("c")
    s = lax.axis_index("s")
    wid = s * NC + c
    base = wid * ROWS
    pltpu.sync_copy(idx_hbm.at[pl.ds(base, ROWS)], idx_v)
    pltpu.sync_copy(val_hbm.at[pl.ds(base, ROWS)], val_v)

    zeros16 = jnp.zeros((L,), jnp.float32)
    sems = (sem0, sem1)

    def fill_group(b, g0):
        for g in range(G):
            def zero(i, c2):
                dense_v[b, g, pl.ds(i * L, L)] = zeros16
                return c2
            lax.fori_loop(0, M // L, zero, 0, unroll=8)
        for g in range(G):
            gvec = jnp.full((L,), g, jnp.int32)
            r = g0 + g
            for q in range(NQ):
                idx16 = idx_v[r, pl.ds(q * L, L)]
                val16 = val_v[r, pl.ds(q * L, L)]
                plsc.addupdate_scatter(dense_v.at[b], [gvec, idx16], val16)
        pltpu.async_copy(
            dense_v.at[b], out_hbm.at[pl.ds(base + g0, G)], sems[b]
        )

    def wait_group(b, g0):
        pltpu.make_async_copy(
            dense_v.at[b], out_hbm.at[pl.ds(base + g0, G)], sems[b]
        ).wait()

    fill_group(0, 0)
    fill_group(1, G)

    def pair(p, carry):
        g0 = 2 * p * G
        wait_group(0, g0 - 2 * G)
        fill_group(0, g0)
        wait_group(1, g0 - G)
        fill_group(1, g0 + G)
        return carry

    lax.fori_loop(1, NG // 2, pair, 0)
    wait_group(0, (NG - 2) * G)
    wait_group(1, (NG - 1) * G)


_sc_call = functools.partial(
    pl.kernel,
    mesh=plsc.VectorSubcoreMesh(core_axis_name="c", subcore_axis_name="s"),
    out_type=jax.ShapeDtypeStruct((B, M), jnp.float32),
    compiler_params=pltpu.CompilerParams(needs_layout_passes=False),
    scratch_types=[
        pltpu.VMEM((ROWS, NNZ), jnp.int32),
        pltpu.VMEM((ROWS, NNZ), jnp.float32),
        pltpu.VMEM((2, G, M), jnp.float32),
        pltpu.SemaphoreType.DMA,
        pltpu.SemaphoreType.DMA,
    ],
)(_body)


def kernel(indices, values):
    return _sc_call(indices, values)
